# baseline (speedup 1.0000x reference)
import functools

import jax
import jax.numpy as jnp
from jax.experimental import pallas as pl
from jax.experimental.pallas import tpu as pltpu

_EPS = 1e-6
_HW = 49
_BN = 4
_NBUF = 8


def _gem_body(p_ref, x_ref, o_ref, *bufsems):
    p = p_ref[0]
    nb = len(bufsems) // 2
    bufs = bufsems[:nb]
    sems = bufsems[nb:]
    n_chunks = x_ref.shape[1] // _BN

    def _copy(k):
        return pltpu.make_async_copy(
            x_ref.at[:, pl.ds(k * _BN, _BN), :], bufs[k % nb], sems[k % nb])

    def _pow_slab(buf, j):
        return jnp.exp2(jnp.log2(jnp.maximum(buf[j], _EPS)) * p)

    for k0 in range(nb - 1):
        _copy(k0).start()
    for k in range(n_chunks):
        _copy(k).wait()
        buf = bufs[k % nb]
        acc = _pow_slab(buf, 0)
        for j in range(1, _HW):
            acc = acc + _pow_slab(buf, j)
        if k + nb - 1 < n_chunks:
            _copy(k + nb - 1).start()
        m = acc * (1.0 / _HW)
        o_ref[pl.ds(k * _BN, _BN), :] = jnp.exp2(jnp.log2(m) * (1.0 / p))


@jax.jit
def _gem_pool(x, p):
    N, C, H, W = x.shape
    xt = x.transpose(2, 3, 0, 1).reshape(H * W, N, C)
    p_arr = jnp.asarray(p, jnp.float32).reshape(1)

    out = pl.pallas_call(
        _gem_body,
        out_shape=jax.ShapeDtypeStruct((N, C), jnp.float32),
        in_specs=[
            pl.BlockSpec(memory_space=pltpu.SMEM),
            pl.BlockSpec(memory_space=pl.ANY),
        ],
        out_specs=pl.BlockSpec(memory_space=pltpu.VMEM),
        scratch_shapes=(
            [pltpu.VMEM((_HW, _BN, C), jnp.float32)] * _NBUF
            + [pltpu.SemaphoreType.DMA] * _NBUF
        ),
        compiler_params=pltpu.CompilerParams(
            vmem_limit_bytes=60 << 20,
        ),
        cost_estimate=pl.CostEstimate(
            flops=int(2 * N * C * H * W),
            transcendentals=int(2 * N * C * H * W + 2 * N * C),
            bytes_accessed=int(x.size * 4 + N * C * 4),
        ),
    )(p_arr, xt)

    return out.reshape(N, C, 1, 1)


def kernel(x, p):
    return _gem_pool(x, p)


# speedup vs baseline: 1.0046x; 1.0046x over previous
import functools

import jax
import jax.numpy as jnp
from jax.experimental import pallas as pl
from jax.experimental.pallas import tpu as pltpu

_EPS = 1e-6
_HW = 49
_BN = 16
_NBUF = 4


def _gem_body(p_ref, x_ref, o_ref, *bufsems):
    p = p_ref[0]
    nb = len(bufsems) // 2
    bufs = bufsems[:nb]
    sems = bufsems[nb:]
    n_chunks = x_ref.shape[1] // _BN

    def _copy(k):
        return pltpu.make_async_copy(
            x_ref.at[:, pl.ds(k * _BN, _BN), :], bufs[k % nb], sems[k % nb])

    def _pow_slab(buf, j):
        return jnp.exp2(jnp.log2(jnp.maximum(buf[j], _EPS)) * p)

    for k0 in range(nb - 1):
        _copy(k0).start()
    for k in range(n_chunks):
        _copy(k).wait()
        buf = bufs[k % nb]
        acc = _pow_slab(buf, 0)
        for j in range(1, _HW):
            acc = acc + _pow_slab(buf, j)
        if k + nb - 1 < n_chunks:
            _copy(k + nb - 1).start()
        m = acc * (1.0 / _HW)
        o_ref[pl.ds(k * _BN, _BN), :] = jnp.exp2(jnp.log2(m) * (1.0 / p))


@jax.jit
def _gem_pool(x, p):
    N, C, H, W = x.shape
    xt = x.transpose(2, 3, 0, 1).reshape(H * W, N, C)
    p_arr = jnp.asarray(p, jnp.float32).reshape(1)

    out = pl.pallas_call(
        _gem_body,
        out_shape=jax.ShapeDtypeStruct((N, C), jnp.float32),
        in_specs=[
            pl.BlockSpec(memory_space=pltpu.SMEM),
            pl.BlockSpec(memory_space=pl.ANY),
        ],
        out_specs=pl.BlockSpec(memory_space=pltpu.VMEM),
        scratch_shapes=(
            [pltpu.VMEM((_HW, _BN, C), jnp.float32)] * _NBUF
            + [pltpu.SemaphoreType.DMA] * _NBUF
        ),
        compiler_params=pltpu.CompilerParams(
            vmem_limit_bytes=60 << 20,
        ),
        cost_estimate=pl.CostEstimate(
            flops=int(2 * N * C * H * W),
            transcendentals=int(2 * N * C * H * W + 2 * N * C),
            bytes_accessed=int(x.size * 4 + N * C * 4),
        ),
    )(p_arr, xt)

    return out.reshape(N, C, 1, 1)


def kernel(x, p):
    return _gem_pool(x, p)


# speedup vs baseline: 1.0464x; 1.0416x over previous
import functools

import jax
import jax.numpy as jnp
from jax.experimental import pallas as pl
from jax.experimental.pallas import tpu as pltpu

_EPS = 1e-6
_HW = 49
_BN = 8
_NBUF = 5


def _gem_body(p_ref, x_ref, o_ref, *bufsems):
    p = p_ref[0]
    nb = len(bufsems) // 2
    bufs = bufsems[:nb]
    sems = bufsems[nb:]
    n_chunks = x_ref.shape[1] // _BN

    def _copy(k):
        return pltpu.make_async_copy(
            x_ref.at[:, pl.ds(k * _BN, _BN), :], bufs[k % nb], sems[k % nb])

    def _pow_slab(buf, j):
        return jnp.exp2(jnp.log2(jnp.maximum(buf[j], _EPS)) * p)

    for k0 in range(nb - 1):
        _copy(k0).start()
    for k in range(n_chunks):
        _copy(k).wait()
        buf = bufs[k % nb]
        acc = _pow_slab(buf, 0)
        for j in range(1, _HW):
            acc = acc + _pow_slab(buf, j)
        if k + nb - 1 < n_chunks:
            _copy(k + nb - 1).start()
        m = acc * (1.0 / _HW)
        o_ref[pl.ds(k * _BN, _BN), :] = jnp.exp2(jnp.log2(m) * (1.0 / p))


@jax.jit
def _gem_pool(x, p):
    N, C, H, W = x.shape
    xt = x.transpose(2, 3, 0, 1).reshape(H * W, N, C)
    p_arr = jnp.asarray(p, jnp.float32).reshape(1)

    out = pl.pallas_call(
        _gem_body,
        out_shape=jax.ShapeDtypeStruct((N, C), jnp.float32),
        in_specs=[
            pl.BlockSpec(memory_space=pltpu.SMEM),
            pl.BlockSpec(memory_space=pl.ANY),
        ],
        out_specs=pl.BlockSpec(memory_space=pltpu.VMEM),
        scratch_shapes=(
            [pltpu.VMEM((_HW, _BN, C), jnp.float32)] * _NBUF
            + [pltpu.SemaphoreType.DMA] * _NBUF
        ),
        compiler_params=pltpu.CompilerParams(
            vmem_limit_bytes=60 << 20,
        ),
        cost_estimate=pl.CostEstimate(
            flops=int(2 * N * C * H * W),
            transcendentals=int(2 * N * C * H * W + 2 * N * C),
            bytes_accessed=int(x.size * 4 + N * C * 4),
        ),
    )(p_arr, xt)

    return out.reshape(N, C, 1, 1)


def kernel(x, p):
    return _gem_pool(x, p)


# speedup vs baseline: 1.0638x; 1.0167x over previous
import functools

import jax
import jax.numpy as jnp
from jax.experimental import pallas as pl
from jax.experimental.pallas import tpu as pltpu

_EPS = 1e-6
_HW = 49
_BN = 8
_NBUF = 4


def _gem_body(p_ref, x_ref, o_ref, *bufsems):
    p = p_ref[0]
    nb = len(bufsems) // 2
    bufs = bufsems[:nb]
    sems = bufsems[nb:]
    n_chunks = x_ref.shape[1] // _BN

    def _copy(k):
        return pltpu.make_async_copy(
            x_ref.at[:, pl.ds(k * _BN, _BN), :], bufs[k % nb], sems[k % nb])

    def _pow_slab(buf, j):
        return jnp.exp2(jnp.log2(jnp.maximum(buf[j], _EPS)) * p)

    for k0 in range(nb - 1):
        _copy(k0).start()
    for k in range(n_chunks):
        _copy(k).wait()
        if k + nb - 1 < n_chunks:
            _copy(k + nb - 1).start()
        buf = bufs[k % nb]
        acc = _pow_slab(buf, 0)
        for j in range(1, _HW):
            acc = acc + _pow_slab(buf, j)
        m = acc * (1.0 / _HW)
        o_ref[pl.ds(k * _BN, _BN), :] = jnp.exp2(jnp.log2(m) * (1.0 / p))


@jax.jit
def _gem_pool(x, p):
    N, C, H, W = x.shape
    xt = x.transpose(2, 3, 0, 1).reshape(H * W, N, C)
    p_arr = jnp.asarray(p, jnp.float32).reshape(1)

    out = pl.pallas_call(
        _gem_body,
        out_shape=jax.ShapeDtypeStruct((N, C), jnp.float32),
        in_specs=[
            pl.BlockSpec(memory_space=pltpu.SMEM),
            pl.BlockSpec(memory_space=pl.ANY),
        ],
        out_specs=pl.BlockSpec(memory_space=pltpu.VMEM),
        scratch_shapes=(
            [pltpu.VMEM((_HW, _BN, C), jnp.float32)] * _NBUF
            + [pltpu.SemaphoreType.DMA] * _NBUF
        ),
        compiler_params=pltpu.CompilerParams(
            vmem_limit_bytes=60 << 20,
        ),
        cost_estimate=pl.CostEstimate(
            flops=int(2 * N * C * H * W),
            transcendentals=int(2 * N * C * H * W + 2 * N * C),
            bytes_accessed=int(x.size * 4 + N * C * 4),
        ),
    )(p_arr, xt)

    return out.reshape(N, C, 1, 1)


def kernel(x, p):
    return _gem_pool(x, p)
